# baseline (speedup 1.0000x reference)
import jax
import jax.numpy as jnp
from jax.experimental import pallas as pl
from jax.experimental.pallas import tpu as pltpu

F32 = jnp.float32
BF16 = jnp.bfloat16


def _mlp_tile(x, w1_ref, b1_ref, g_ref, be_ref, w2_ref, b2_ref):
    h = (jnp.dot(x.astype(BF16), w1_ref[...], preferred_element_type=F32)
         + b1_ref[...])
    mu = jnp.mean(h, axis=-1, keepdims=True)
    var = jnp.mean((h - mu) ** 2, axis=-1, keepdims=True)
    h = (h - mu) * jax.lax.rsqrt(var + 1e-5) * g_ref[...] + be_ref[...]
    h = 0.5 * h * (1.0 + jax.lax.erf(h * 0.7071067811865476))
    return (jnp.dot(h.astype(BF16), w2_ref[...], preferred_element_type=F32)
            + b2_ref[...])


def _make_entity_body(nwc, nwt, na, nt, bw, ba, bt2, br):
    def body(sc_ref, wc_ref, wt_ref, adj_c2_ref, adj_t2_ref,
             adj_c1_ref, adj_t1_ref,
             w1_ref, b1_ref, g_ref, be_ref, w2_ref, b2_ref,
             pw_ref, pb_ref,
             emb_ref, log_ref, pc_ref, pt_ref,
             wcbf, wtbf, tc_scr, tt_scr, w1s, w2s, pws):
        i = pl.program_id(0)

        @pl.when(i == 0)
        def _():
            w1s[...] = w1_ref[...].astype(BF16)
            w2s[...] = w2_ref[...].astype(BF16)
            pws[...] = pw_ref[...].astype(BF16)

        @pl.when(i < nwc)
        def _():
            wcbf[pl.ds(i * bw, bw), :] = wc_ref[...].astype(BF16)

        j = i - nwc

        @pl.when((i >= nwc) & (j < nwt))
        def _():
            wtbf[pl.ds(j * bw, bw), :] = wt_ref[...].astype(BF16)

        @pl.when((i >= nwc) & (i < nwc + na))
        def _():
            t = jnp.dot(adj_c2_ref[...].astype(BF16), wcbf[...],
                        preferred_element_type=F32)
            tc_scr[pl.ds(j * ba, ba), :] = t.astype(BF16)

        q = i - nwc - na

        @pl.when((i >= nwc + na) & (i < nwc + na + nt))
        def _():
            t = jnp.dot(adj_t2_ref[...].astype(BF16), wtbf[...],
                        preferred_element_type=F32)
            tt_scr[pl.ds(q * bt2, bt2), :] = t.astype(BF16)

        @pl.when(i >= nwc + na + nt)
        def _():
            r = i - nwc - na - nt
            wc_rows = wcbf[pl.ds(r * br, br), :].astype(F32)
            wt_rows = wtbf[pl.ds(r * br, br), :].astype(F32)
            emb_c = sc_ref[0] * wc_rows + sc_ref[1] * jnp.dot(
                adj_c1_ref[...].astype(BF16), tc_scr[...],
                preferred_element_type=F32)
            emb_t = sc_ref[2] * wt_rows + sc_ref[3] * jnp.dot(
                adj_t1_ref[...].astype(BF16), tt_scr[...],
                preferred_element_type=F32)
            e = sc_ref[4] * emb_c + sc_ref[5] * emb_t
            emb_ref[...] = e
            log_ref[...] = (
                jnp.dot(e.astype(BF16), pws[...], preferred_element_type=F32)
                + pb_ref[...])
            pc_ref[...] = _mlp_tile(emb_c, w1s, b1_ref, g_ref, be_ref, w2s,
                                    b2_ref)
            pt_ref[...] = _mlp_tile(emb_t, w1s, b1_ref, g_ref, be_ref, w2s,
                                    b2_ref)

    return body


def _entity_fused(scalars, adj_c2, adj_t2, w_c, w_t, adj_c1, adj_t1,
                  fc1_w, fc1_b, ln_g, ln_b, fc2_w, fc2_b, pred_w, pred_b,
                  bw, ba, bt2, br):
    m, kc = adj_c1.shape
    ktc = adj_c2.shape[1]
    ktt = adj_t2.shape[1]
    kt1 = adj_t1.shape[1]
    d = w_c.shape[1]
    t = pred_w.shape[1]
    nwc = w_c.shape[0] // bw
    nwt = w_t.shape[0] // bw
    na = adj_c2.shape[0] // ba
    nt = adj_t2.shape[0] // bt2
    nr = m // br
    assert nwt <= na

    const = lambda shape: pl.BlockSpec(shape, lambda i: (0, 0))
    wc_map = lambda i: (jnp.minimum(i, nwc - 1), 0)
    wt_map = lambda i: (jnp.clip(i - nwc, 0, nwt - 1), 0)
    a_map = lambda i: (jnp.clip(i - nwc, 0, na - 1), 0)
    t_map = lambda i: (jnp.clip(i - nwc - na, 0, nt - 1), 0)
    r_map = lambda i: (jnp.clip(i - nwc - na - nt, 0, nr - 1), 0)

    return pl.pallas_call(
        _make_entity_body(nwc, nwt, na, nt, bw, ba, bt2, br),
        grid=(nwc + na + nt + nr,),
        in_specs=[
            pl.BlockSpec(memory_space=pltpu.SMEM),
            pl.BlockSpec((bw, d), wc_map),
            pl.BlockSpec((bw, d), wt_map),
            pl.BlockSpec((ba, ktc), a_map),
            pl.BlockSpec((bt2, ktt), t_map),
            pl.BlockSpec((br, kc), r_map),
            pl.BlockSpec((br, kt1), r_map),
            const((d, d)), const((1, d)), const((1, d)), const((1, d)),
            const((d, d)), const((1, d)),
            const((d, t)), const((1, t)),
        ],
        out_specs=[
            pl.BlockSpec((br, d), r_map),
            pl.BlockSpec((br, t), r_map),
            pl.BlockSpec((br, d), r_map),
            pl.BlockSpec((br, d), r_map),
        ],
        out_shape=[
            jax.ShapeDtypeStruct((m, d), F32),
            jax.ShapeDtypeStruct((m, t), F32),
            jax.ShapeDtypeStruct((m, d), F32),
            jax.ShapeDtypeStruct((m, d), F32),
        ],
        scratch_shapes=[
            pltpu.VMEM((w_c.shape[0], d), BF16),
            pltpu.VMEM((w_t.shape[0], d), BF16),
            pltpu.VMEM((adj_c2.shape[0], d), BF16),
            pltpu.VMEM((adj_t2.shape[0], d), BF16),
            pltpu.VMEM((d, d), BF16),
            pltpu.VMEM((d, d), BF16),
            pltpu.VMEM((d, t), BF16),
        ],
        compiler_params=pltpu.CompilerParams(
            dimension_semantics=("arbitrary",),
            vmem_limit_bytes=67000000),
    )(scalars, w_c, w_t, adj_c2, adj_t2, adj_c1, adj_t1,
      fc1_w, fc1_b, ln_g, ln_b, fc2_w, fc2_b, pred_w, pred_b)


def kernel(adj_m_c1, adj_m_c2, adj_a_c1, adj_a_c2, adj_m_t1, adj_m_t2,
           adj_a_t1, adj_a_t2, mashup_call_W, api_call_W, mashup_tag_W,
           api_tag_W, u_weights, i_weights, m_t_weights, a_t_weights,
           mashup_view_weights, api_view_weights, m_fc1_w, m_fc1_b,
           m_ln_g, m_ln_b, m_fc2_w, m_fc2_b, a_fc1_w, a_fc1_b, a_ln_g,
           a_ln_b, a_fc2_w, a_fc2_b, m_pred_w, m_pred_b, a_pred_w,
           a_pred_b):
    uw = jax.nn.softmax(u_weights, axis=0)
    iw = jax.nn.softmax(i_weights, axis=0)
    mtw = jax.nn.softmax(m_t_weights, axis=0)
    atw = jax.nn.softmax(a_t_weights, axis=0)
    mvw = jax.nn.softmax(mashup_view_weights, axis=0)
    avw = jax.nn.softmax(api_view_weights, axis=0)
    m_scal = jnp.concatenate([uw, mtw, mvw])
    a_scal = jnp.concatenate([iw, atw, avw])

    mashup_emb, m_logits, mashup_call_proj, mashup_tag_proj = _entity_fused(
        m_scal, adj_m_c2, adj_m_t2, mashup_call_W, mashup_tag_W,
        adj_m_c1, adj_m_t1,
        m_fc1_w, m_fc1_b.reshape(1, -1),
        m_ln_g.reshape(1, -1), m_ln_b.reshape(1, -1),
        m_fc2_w, m_fc2_b.reshape(1, -1),
        m_pred_w, m_pred_b.reshape(1, -1),
        bw=256, ba=128, bt2=128, br=512)
    api_emb, a_logits, api_call_proj, api_tag_proj = _entity_fused(
        a_scal, adj_a_c2, adj_a_t2, api_call_W, api_tag_W,
        adj_a_c1, adj_a_t1,
        a_fc1_w, a_fc1_b.reshape(1, -1),
        a_ln_g.reshape(1, -1), a_ln_b.reshape(1, -1),
        a_fc2_w, a_fc2_b.reshape(1, -1),
        a_pred_w, a_pred_b.reshape(1, -1),
        bw=256, ba=128, bt2=128, br=512)

    return (mashup_emb, api_emb, mashup_call_proj, mashup_tag_proj,
            api_call_proj, api_tag_proj, m_logits, a_logits)


# speedup vs baseline: 1.0651x; 1.0651x over previous
import jax
import jax.numpy as jnp
from jax.experimental import pallas as pl
from jax.experimental.pallas import tpu as pltpu

F32 = jnp.float32
BF16 = jnp.bfloat16


def _mlp_tile(x, w1_ref, b1_ref, g_ref, be_ref, w2_ref, b2_ref):
    h = (jnp.dot(x.astype(BF16), w1_ref[...], preferred_element_type=F32)
         + b1_ref[...])
    mu = jnp.mean(h, axis=-1, keepdims=True)
    var = jnp.mean((h - mu) ** 2, axis=-1, keepdims=True)
    h = (h - mu) * jax.lax.rsqrt(var + 1e-5) * g_ref[...] + be_ref[...]
    h = 0.5 * h * (1.0 + jax.lax.erf(h * 0.7071067811865476))
    return (jnp.dot(h.astype(BF16), w2_ref[...], preferred_element_type=F32)
            + b2_ref[...])


def _make_entity_body(nwc, nwt, na, nt, bw, ba, bt2, br):
    def body(sc_ref, wc_ref, wt_ref, adj_c2_ref, adj_t2_ref,
             adj_c1_ref, adj_t1_ref,
             w1_ref, b1_ref, g_ref, be_ref, w2_ref, b2_ref,
             pw_ref, pb_ref,
             emb_ref, log_ref, pc_ref, pt_ref,
             wcbf, wtbf, tc_scr, tt_scr, w1s, w2s, pws):
        i = pl.program_id(0)

        @pl.when(i == 0)
        def _():
            w1s[...] = w1_ref[...].astype(BF16)
            w2s[...] = w2_ref[...].astype(BF16)
            pws[...] = pw_ref[...].astype(BF16)

        @pl.when(i < nwc)
        def _():
            wcbf[pl.ds(i * bw, bw), :] = wc_ref[...].astype(BF16)

        j = i - nwc

        @pl.when((i >= nwc) & (j < nwt))
        def _():
            wtbf[pl.ds(j * bw, bw), :] = wt_ref[...].astype(BF16)

        @pl.when((i >= nwc) & (i < nwc + na))
        def _():
            t = jnp.dot(adj_c2_ref[...].astype(BF16), wcbf[...],
                        preferred_element_type=F32)
            tc_scr[pl.ds(j * ba, ba), :] = t.astype(BF16)

        q = i - nwc - na

        @pl.when((i >= nwc + na) & (i < nwc + na + nt))
        def _():
            t = jnp.dot(adj_t2_ref[...].astype(BF16), wtbf[...],
                        preferred_element_type=F32)
            tt_scr[pl.ds(q * bt2, bt2), :] = t.astype(BF16)

        @pl.when(i >= nwc + na + nt)
        def _():
            r = i - nwc - na - nt
            wc_rows = wcbf[pl.ds(r * br, br), :].astype(F32)
            wt_rows = wtbf[pl.ds(r * br, br), :].astype(F32)
            emb_c = sc_ref[0] * wc_rows + sc_ref[1] * jnp.dot(
                adj_c1_ref[...].astype(BF16), tc_scr[...],
                preferred_element_type=F32)
            emb_t = sc_ref[2] * wt_rows + sc_ref[3] * jnp.dot(
                adj_t1_ref[...].astype(BF16), tt_scr[...],
                preferred_element_type=F32)
            e = sc_ref[4] * emb_c + sc_ref[5] * emb_t
            emb_ref[...] = e
            log_ref[...] = (
                jnp.dot(e.astype(BF16), pws[...], preferred_element_type=F32)
                + pb_ref[...])
            pc_ref[...] = _mlp_tile(emb_c, w1s, b1_ref, g_ref, be_ref, w2s,
                                    b2_ref)
            pt_ref[...] = _mlp_tile(emb_t, w1s, b1_ref, g_ref, be_ref, w2s,
                                    b2_ref)

    return body


def _entity_fused(scalars, adj_c2, adj_t2, w_c, w_t, adj_c1, adj_t1,
                  fc1_w, fc1_b, ln_g, ln_b, fc2_w, fc2_b, pred_w, pred_b,
                  bw, ba, bt2, br):
    m, kc = adj_c1.shape
    ktc = adj_c2.shape[1]
    ktt = adj_t2.shape[1]
    kt1 = adj_t1.shape[1]
    d = w_c.shape[1]
    t = pred_w.shape[1]
    nwc = w_c.shape[0] // bw
    nwt = w_t.shape[0] // bw
    na = adj_c2.shape[0] // ba
    nt = adj_t2.shape[0] // bt2
    nr = m // br
    assert nwt <= na

    const = lambda shape: pl.BlockSpec(shape, lambda i: (0, 0))
    wc_map = lambda i: (jnp.minimum(i, nwc - 1), 0)
    wt_map = lambda i: (jnp.clip(i - nwc, 0, nwt - 1), 0)
    a_map = lambda i: (jnp.clip(i - nwc, 0, na - 1), 0)
    t_map = lambda i: (jnp.clip(i - nwc - na, 0, nt - 1), 0)
    r_map = lambda i: (jnp.clip(i - nwc - na - nt, 0, nr - 1), 0)

    return pl.pallas_call(
        _make_entity_body(nwc, nwt, na, nt, bw, ba, bt2, br),
        grid=(nwc + na + nt + nr,),
        in_specs=[
            pl.BlockSpec(memory_space=pltpu.SMEM),
            pl.BlockSpec((bw, d), wc_map),
            pl.BlockSpec((bw, d), wt_map),
            pl.BlockSpec((ba, ktc), a_map),
            pl.BlockSpec((bt2, ktt), t_map),
            pl.BlockSpec((br, kc), r_map),
            pl.BlockSpec((br, kt1), r_map),
            const((d, d)), const((1, d)), const((1, d)), const((1, d)),
            const((d, d)), const((1, d)),
            const((d, t)), const((1, t)),
        ],
        out_specs=[
            pl.BlockSpec((br, d), r_map),
            pl.BlockSpec((br, t), r_map),
            pl.BlockSpec((br, d), r_map),
            pl.BlockSpec((br, d), r_map),
        ],
        out_shape=[
            jax.ShapeDtypeStruct((m, d), F32),
            jax.ShapeDtypeStruct((m, t), F32),
            jax.ShapeDtypeStruct((m, d), F32),
            jax.ShapeDtypeStruct((m, d), F32),
        ],
        scratch_shapes=[
            pltpu.VMEM((w_c.shape[0], d), BF16),
            pltpu.VMEM((w_t.shape[0], d), BF16),
            pltpu.VMEM((adj_c2.shape[0], d), BF16),
            pltpu.VMEM((adj_t2.shape[0], d), BF16),
            pltpu.VMEM((d, d), BF16),
            pltpu.VMEM((d, d), BF16),
            pltpu.VMEM((d, t), BF16),
        ],
        compiler_params=pltpu.CompilerParams(
            dimension_semantics=("arbitrary",),
            vmem_limit_bytes=67000000),
    )(scalars, w_c, w_t, adj_c2, adj_t2, adj_c1, adj_t1,
      fc1_w, fc1_b, ln_g, ln_b, fc2_w, fc2_b, pred_w, pred_b)


def kernel(adj_m_c1, adj_m_c2, adj_a_c1, adj_a_c2, adj_m_t1, adj_m_t2,
           adj_a_t1, adj_a_t2, mashup_call_W, api_call_W, mashup_tag_W,
           api_tag_W, u_weights, i_weights, m_t_weights, a_t_weights,
           mashup_view_weights, api_view_weights, m_fc1_w, m_fc1_b,
           m_ln_g, m_ln_b, m_fc2_w, m_fc2_b, a_fc1_w, a_fc1_b, a_ln_g,
           a_ln_b, a_fc2_w, a_fc2_b, m_pred_w, m_pred_b, a_pred_w,
           a_pred_b):
    uw = jax.nn.softmax(u_weights, axis=0)
    iw = jax.nn.softmax(i_weights, axis=0)
    mtw = jax.nn.softmax(m_t_weights, axis=0)
    atw = jax.nn.softmax(a_t_weights, axis=0)
    mvw = jax.nn.softmax(mashup_view_weights, axis=0)
    avw = jax.nn.softmax(api_view_weights, axis=0)
    m_scal = jnp.concatenate([uw, mtw, mvw])
    a_scal = jnp.concatenate([iw, atw, avw])

    mashup_emb, m_logits, mashup_call_proj, mashup_tag_proj = _entity_fused(
        m_scal, adj_m_c2, adj_m_t2, mashup_call_W, mashup_tag_W,
        adj_m_c1, adj_m_t1,
        m_fc1_w, m_fc1_b.reshape(1, -1),
        m_ln_g.reshape(1, -1), m_ln_b.reshape(1, -1),
        m_fc2_w, m_fc2_b.reshape(1, -1),
        m_pred_w, m_pred_b.reshape(1, -1),
        bw=512, ba=256, bt2=128, br=256)
    api_emb, a_logits, api_call_proj, api_tag_proj = _entity_fused(
        a_scal, adj_a_c2, adj_a_t2, api_call_W, api_tag_W,
        adj_a_c1, adj_a_t1,
        a_fc1_w, a_fc1_b.reshape(1, -1),
        a_ln_g.reshape(1, -1), a_ln_b.reshape(1, -1),
        a_fc2_w, a_fc2_b.reshape(1, -1),
        a_pred_w, a_pred_b.reshape(1, -1),
        bw=512, ba=256, bt2=128, br=256)

    return (mashup_emb, api_emb, mashup_call_proj, mashup_tag_proj,
            api_call_proj, api_tag_proj, m_logits, a_logits)


# speedup vs baseline: 1.0865x; 1.0200x over previous
import jax
import jax.numpy as jnp
from jax.experimental import pallas as pl
from jax.experimental.pallas import tpu as pltpu

F32 = jnp.float32
BF16 = jnp.bfloat16


def _mlp_tile(x, w1_ref, b1_ref, g_ref, be_ref, w2_ref, b2_ref):
    h = (jnp.dot(x.astype(BF16), w1_ref[...], preferred_element_type=F32)
         + b1_ref[...])
    mu = jnp.mean(h, axis=-1, keepdims=True)
    var = jnp.mean((h - mu) ** 2, axis=-1, keepdims=True)
    h = (h - mu) * jax.lax.rsqrt(var + 1e-5) * g_ref[...] + be_ref[...]
    h = 0.5 * h * (1.0 + jax.lax.erf(h * 0.7071067811865476))
    return (jnp.dot(h.astype(BF16), w2_ref[...], preferred_element_type=F32)
            + b2_ref[...])


def _make_entity_body(nwc, nwt, na, nt, bw, ba, bt2, br):
    def body(sc_ref, wc_ref, wt_ref, adj_c2_ref, adj_t2_ref,
             adj_c1_ref, adj_t1_ref,
             w1_ref, b1_ref, g_ref, be_ref, w2_ref, b2_ref,
             pw_ref, pb_ref,
             emb_ref, log_ref, pc_ref, pt_ref,
             wcbf, wtbf, tc_scr, tt_scr, w1s, w2s, pws):
        i = pl.program_id(0)

        @pl.when(i == 0)
        def _():
            w1s[...] = w1_ref[...].astype(BF16)
            w2s[...] = w2_ref[...].astype(BF16)
            pws[...] = pw_ref[...].astype(BF16)

        @pl.when(i < nwc)
        def _():
            wcbf[pl.ds(i * bw, bw), :] = wc_ref[...].astype(BF16)

        j = i - nwc

        @pl.when((i >= nwc) & (j < nwt))
        def _():
            wtbf[pl.ds(j * bw, bw), :] = wt_ref[...].astype(BF16)

        @pl.when((i >= nwc) & (i < nwc + na))
        def _():
            t = jnp.dot(adj_c2_ref[...].astype(BF16), wcbf[...],
                        preferred_element_type=F32)
            tc_scr[pl.ds(j * ba, ba), :] = t.astype(BF16)

        q = i - nwc - na

        @pl.when((i >= nwc + na) & (i < nwc + na + nt))
        def _():
            t = jnp.dot(adj_t2_ref[...].astype(BF16), wtbf[...],
                        preferred_element_type=F32)
            tt_scr[pl.ds(q * bt2, bt2), :] = t.astype(BF16)

        @pl.when(i >= nwc + na + nt)
        def _():
            r = i - nwc - na - nt
            wc_rows = wcbf[pl.ds(r * br, br), :].astype(F32)
            wt_rows = wtbf[pl.ds(r * br, br), :].astype(F32)
            emb_c = sc_ref[0] * wc_rows + sc_ref[1] * jnp.dot(
                adj_c1_ref[...].astype(BF16), tc_scr[...],
                preferred_element_type=F32)
            emb_t = sc_ref[2] * wt_rows + sc_ref[3] * jnp.dot(
                adj_t1_ref[...].astype(BF16), tt_scr[...],
                preferred_element_type=F32)
            e = sc_ref[4] * emb_c + sc_ref[5] * emb_t
            emb_ref[...] = e
            log_ref[...] = (
                jnp.dot(e.astype(BF16), pws[...], preferred_element_type=F32)
                + pb_ref[...])
            pc_ref[...] = _mlp_tile(emb_c, w1s, b1_ref, g_ref, be_ref, w2s,
                                    b2_ref)
            pt_ref[...] = _mlp_tile(emb_t, w1s, b1_ref, g_ref, be_ref, w2s,
                                    b2_ref)

    return body


def _entity_fused(scalars, adj_c2, adj_t2, w_c, w_t, adj_c1, adj_t1,
                  fc1_w, fc1_b, ln_g, ln_b, fc2_w, fc2_b, pred_w, pred_b,
                  bw, ba, bt2, br):
    m, kc = adj_c1.shape
    ktc = adj_c2.shape[1]
    ktt = adj_t2.shape[1]
    kt1 = adj_t1.shape[1]
    d = w_c.shape[1]
    t = pred_w.shape[1]
    nwc = w_c.shape[0] // bw
    nwt = w_t.shape[0] // bw
    na = adj_c2.shape[0] // ba
    nt = adj_t2.shape[0] // bt2
    nr = m // br
    assert nwt <= na

    const = lambda shape: pl.BlockSpec(shape, lambda i: (0, 0))
    wc_map = lambda i: (jnp.minimum(i, nwc - 1), 0)
    wt_map = lambda i: (jnp.clip(i - nwc, 0, nwt - 1), 0)
    a_map = lambda i: (jnp.clip(i - nwc, 0, na - 1), 0)
    t_map = lambda i: (jnp.clip(i - nwc - na, 0, nt - 1), 0)
    r_map = lambda i: (jnp.clip(i - nwc - na - nt, 0, nr - 1), 0)

    return pl.pallas_call(
        _make_entity_body(nwc, nwt, na, nt, bw, ba, bt2, br),
        grid=(nwc + na + nt + nr,),
        in_specs=[
            pl.BlockSpec(memory_space=pltpu.SMEM),
            pl.BlockSpec((bw, d), wc_map),
            pl.BlockSpec((bw, d), wt_map),
            pl.BlockSpec((ba, ktc), a_map),
            pl.BlockSpec((bt2, ktt), t_map),
            pl.BlockSpec((br, kc), r_map),
            pl.BlockSpec((br, kt1), r_map),
            const((d, d)), const((1, d)), const((1, d)), const((1, d)),
            const((d, d)), const((1, d)),
            const((d, t)), const((1, t)),
        ],
        out_specs=[
            pl.BlockSpec((br, d), r_map),
            pl.BlockSpec((br, t), r_map),
            pl.BlockSpec((br, d), r_map),
            pl.BlockSpec((br, d), r_map),
        ],
        out_shape=[
            jax.ShapeDtypeStruct((m, d), F32),
            jax.ShapeDtypeStruct((m, t), F32),
            jax.ShapeDtypeStruct((m, d), F32),
            jax.ShapeDtypeStruct((m, d), F32),
        ],
        scratch_shapes=[
            pltpu.VMEM((w_c.shape[0], d), BF16),
            pltpu.VMEM((w_t.shape[0], d), BF16),
            pltpu.VMEM((adj_c2.shape[0], d), BF16),
            pltpu.VMEM((adj_t2.shape[0], d), BF16),
            pltpu.VMEM((d, d), BF16),
            pltpu.VMEM((d, d), BF16),
            pltpu.VMEM((d, t), BF16),
        ],
        compiler_params=pltpu.CompilerParams(
            dimension_semantics=("arbitrary",),
            vmem_limit_bytes=67000000),
    )(scalars, w_c, w_t, adj_c2, adj_t2, adj_c1, adj_t1,
      fc1_w, fc1_b, ln_g, ln_b, fc2_w, fc2_b, pred_w, pred_b)


def kernel(adj_m_c1, adj_m_c2, adj_a_c1, adj_a_c2, adj_m_t1, adj_m_t2,
           adj_a_t1, adj_a_t2, mashup_call_W, api_call_W, mashup_tag_W,
           api_tag_W, u_weights, i_weights, m_t_weights, a_t_weights,
           mashup_view_weights, api_view_weights, m_fc1_w, m_fc1_b,
           m_ln_g, m_ln_b, m_fc2_w, m_fc2_b, a_fc1_w, a_fc1_b, a_ln_g,
           a_ln_b, a_fc2_w, a_fc2_b, m_pred_w, m_pred_b, a_pred_w,
           a_pred_b):
    uw = jax.nn.softmax(u_weights, axis=0)
    iw = jax.nn.softmax(i_weights, axis=0)
    mtw = jax.nn.softmax(m_t_weights, axis=0)
    atw = jax.nn.softmax(a_t_weights, axis=0)
    mvw = jax.nn.softmax(mashup_view_weights, axis=0)
    avw = jax.nn.softmax(api_view_weights, axis=0)
    m_scal = jnp.concatenate([uw, mtw, mvw])
    a_scal = jnp.concatenate([iw, atw, avw])

    mashup_emb, m_logits, mashup_call_proj, mashup_tag_proj = _entity_fused(
        m_scal, adj_m_c2, adj_m_t2, mashup_call_W, mashup_tag_W,
        adj_m_c1, adj_m_t1,
        m_fc1_w, m_fc1_b.reshape(1, -1),
        m_ln_g.reshape(1, -1), m_ln_b.reshape(1, -1),
        m_fc2_w, m_fc2_b.reshape(1, -1),
        m_pred_w, m_pred_b.reshape(1, -1),
        bw=512, ba=256, bt2=128, br=256)
    api_emb, a_logits, api_call_proj, api_tag_proj = _entity_fused(
        a_scal, adj_a_c2, adj_a_t2, api_call_W, api_tag_W,
        adj_a_c1, adj_a_t1,
        a_fc1_w, a_fc1_b.reshape(1, -1),
        a_ln_g.reshape(1, -1), a_ln_b.reshape(1, -1),
        a_fc2_w, a_fc2_b.reshape(1, -1),
        a_pred_w, a_pred_b.reshape(1, -1),
        bw=512, ba=512, bt2=128, br=256)

    return (mashup_emb, api_emb, mashup_call_proj, mashup_tag_proj,
            api_call_proj, api_tag_proj, m_logits, a_logits)
